# baseline (speedup 1.0000x reference)
import functools

import jax
import jax.numpy as jnp
import numpy as np
from jax.experimental import pallas as pl
from jax.experimental.pallas import tpu as pltpu

S = 2048
DM = 2048
H = 12
D = 64
RD = 32
TOPK = 512
QBLK = 256


def _proj_body(hs_ref, wq_ref, wqs_ref, wk_ref, wks_ref, ww_ref,
               c2h_ref, s2h_ref, c2_ref, s2_ref, q_ref, k_ref, w_ref):
    hsb = hs_ref[...]
    q1 = jnp.dot(hsb, wq_ref[...], preferred_element_type=jnp.float32)
    q2 = jnp.dot(hsb, wqs_ref[...], preferred_element_type=jnp.float32)
    q_ref[...] = q1 * c2h_ref[...] + q2 * s2h_ref[...]
    k1 = jnp.dot(hsb, wk_ref[...], preferred_element_type=jnp.float32)
    k2 = jnp.dot(hsb, wks_ref[...], preferred_element_type=jnp.float32)
    k_ref[...] = k1 * c2_ref[...] + k2 * s2_ref[...]
    g = jnp.dot(hsb, ww_ref[...], preferred_element_type=jnp.float32)
    g = g - jnp.max(g, axis=-1, keepdims=True)
    e = jnp.exp(g)
    w_ref[...] = e / jnp.sum(e, axis=-1, keepdims=True)


def _scores_body(q_ref, w_ref, k_ref, out_ref):
    i = pl.program_id(0)
    q = q_ref[...]
    k = k_ref[...]
    w = w_ref[...]
    acc = jnp.zeros((QBLK, S), dtype=jnp.float32)
    for h in range(H):
        qh = q[:, h * D:(h + 1) * D]
        lh = jax.lax.dot_general(qh, k, (((1,), (1,)), ((), ())),
                                 preferred_element_type=jnp.float32)
        acc = acc + jnp.maximum(lh * (D ** -0.5), 0.0) * w[:, h:h + 1]
    row = i * QBLK + jax.lax.broadcasted_iota(jnp.int32, (QBLK, S), 0)
    col = jax.lax.broadcasted_iota(jnp.int32, (QBLK, S), 1)
    out_ref[...] = jnp.where(col <= row, acc, -1e9)


def _compute_scores(hs, cos, sin, wq, wk, ww):
    c = cos[:, :RD // 2]
    sn = sin[:, :RD // 2]
    c2r = jnp.repeat(c, 2, axis=-1)
    s2r = jnp.repeat(sn, 2, axis=-1)
    sgn = jnp.tile(jnp.array([-1.0, 1.0], dtype=jnp.float32), RD // 2)
    ones = jnp.ones((S, D - RD), dtype=jnp.float32)
    zeros = jnp.zeros((S, D - RD), dtype=jnp.float32)
    c2 = jnp.concatenate([ones, c2r], axis=-1)
    s2 = jnp.concatenate([zeros, s2r * sgn], axis=-1)
    c2h = jnp.tile(c2, (1, H))
    s2h = jnp.tile(s2, (1, H))
    perm = np.arange(D)
    perm[D - RD:] = perm[D - RD:] ^ 1
    wq_sw = wq.reshape(DM, H, D)[:, :, perm].reshape(DM, H * D)
    wk_sw = wk[:, perm]

    grid = S // QBLK
    q, k, w = pl.pallas_call(
        _proj_body,
        grid=(grid,),
        in_specs=[
            pl.BlockSpec((QBLK, DM), lambda i: (i, 0)),
            pl.BlockSpec((DM, H * D), lambda i: (0, 0)),
            pl.BlockSpec((DM, H * D), lambda i: (0, 0)),
            pl.BlockSpec((DM, D), lambda i: (0, 0)),
            pl.BlockSpec((DM, D), lambda i: (0, 0)),
            pl.BlockSpec((DM, H), lambda i: (0, 0)),
            pl.BlockSpec((QBLK, H * D), lambda i: (i, 0)),
            pl.BlockSpec((QBLK, H * D), lambda i: (i, 0)),
            pl.BlockSpec((QBLK, D), lambda i: (i, 0)),
            pl.BlockSpec((QBLK, D), lambda i: (i, 0)),
        ],
        out_specs=[
            pl.BlockSpec((QBLK, H * D), lambda i: (i, 0)),
            pl.BlockSpec((QBLK, D), lambda i: (i, 0)),
            pl.BlockSpec((QBLK, H), lambda i: (i, 0)),
        ],
        out_shape=[
            jax.ShapeDtypeStruct((S, H * D), jnp.float32),
            jax.ShapeDtypeStruct((S, D), jnp.float32),
            jax.ShapeDtypeStruct((S, H), jnp.float32),
        ],
    )(hs, wq, wq_sw, wk, wk_sw, ww, c2h, s2h, c2, s2)

    scores = pl.pallas_call(
        _scores_body,
        grid=(grid,),
        in_specs=[
            pl.BlockSpec((QBLK, H * D), lambda i: (i, 0)),
            pl.BlockSpec((QBLK, H), lambda i: (i, 0)),
            pl.BlockSpec((S, D), lambda i: (0, 0)),
        ],
        out_specs=pl.BlockSpec((QBLK, S), lambda i: (i, 0)),
        out_shape=jax.ShapeDtypeStruct((S, S), jnp.float32),
    )(q, w, k)
    return scores


@jax.jit
def kernel(hidden_states, cos, sin, wq, wk, ww):
    hs = hidden_states[0]
    scores = _compute_scores(hs, cos[0], sin[0], wq, wk, ww)
    tv, ti = jax.lax.top_k(scores, TOPK)
    return tv[None], ti[None]


# speedup vs baseline: 2.0050x; 2.0050x over previous
import functools

import jax
import jax.numpy as jnp
import numpy as np
from jax import lax
from jax.experimental import pallas as pl
from jax.experimental.pallas import tpu as pltpu
from jax.experimental.pallas import tpu_sc as plsc

S = 2048
DM = 2048
H = 12
D = 64
RD = 32
TOPK = 512
QBLK = 256

NC = 2
NS = 16
NW = NC * NS
ROWS_PER_W = S // NW
L = 16
CHUNK = 256
NVC = CHUNK // L

MASK_VAL = -1e9
ZERO_KEY_BIAS = 0
MASK_KEY_BIAS = 4096


def _proj_body(hs_ref, wq_ref, wqs_ref, wk_ref, wks_ref, ww_ref,
               c2h_ref, s2h_ref, c2_ref, s2_ref, q_ref, k_ref, w_ref):
    hsb = hs_ref[...]
    q1 = jnp.dot(hsb, wq_ref[...], preferred_element_type=jnp.float32)
    q2 = jnp.dot(hsb, wqs_ref[...], preferred_element_type=jnp.float32)
    q_ref[...] = q1 * c2h_ref[...] + q2 * s2h_ref[...]
    k1 = jnp.dot(hsb, wk_ref[...], preferred_element_type=jnp.float32)
    k2 = jnp.dot(hsb, wks_ref[...], preferred_element_type=jnp.float32)
    k_ref[...] = k1 * c2_ref[...] + k2 * s2_ref[...]
    g = jnp.dot(hsb, ww_ref[...], preferred_element_type=jnp.float32)
    g = g - jnp.max(g, axis=-1, keepdims=True)
    e = jnp.exp(g)
    w_ref[...] = e / jnp.sum(e, axis=-1, keepdims=True)


def _scores_body(q_ref, w_ref, k_ref, out_ref):
    i = pl.program_id(0)
    q = q_ref[...]
    k = k_ref[...]
    w = w_ref[...]
    acc = jnp.zeros((QBLK, S), dtype=jnp.float32)
    for h in range(H):
        qh = q[:, h * D:(h + 1) * D]
        lh = lax.dot_general(qh, k, (((1,), (1,)), ((), ())),
                             preferred_element_type=jnp.float32)
        acc = acc + jnp.maximum(lh * (D ** -0.5), 0.0) * w[:, h:h + 1]
    row = i * QBLK + lax.broadcasted_iota(jnp.int32, (QBLK, S), 0)
    col = lax.broadcasted_iota(jnp.int32, (QBLK, S), 1)
    out_ref[...] = jnp.where(col <= row, acc, MASK_VAL)


def _compute_scores(hs, cos, sin, wq, wk, ww):
    c = cos[:, :RD // 2]
    sn = sin[:, :RD // 2]
    c2r = jnp.repeat(c, 2, axis=-1)
    s2r = jnp.repeat(sn, 2, axis=-1)
    sgn = jnp.tile(jnp.array([-1.0, 1.0], dtype=jnp.float32), RD // 2)
    ones = jnp.ones((S, D - RD), dtype=jnp.float32)
    zeros = jnp.zeros((S, D - RD), dtype=jnp.float32)
    c2 = jnp.concatenate([ones, c2r], axis=-1)
    s2 = jnp.concatenate([zeros, s2r * sgn], axis=-1)
    c2h = jnp.tile(c2, (1, H))
    s2h = jnp.tile(s2, (1, H))
    perm = np.arange(D)
    perm[D - RD:] = perm[D - RD:] ^ 1
    wq_sw = wq.reshape(DM, H, D)[:, :, perm].reshape(DM, H * D)
    wk_sw = wk[:, perm]

    grid = S // QBLK
    q, k, w = pl.pallas_call(
        _proj_body,
        grid=(grid,),
        in_specs=[
            pl.BlockSpec((QBLK, DM), lambda i: (i, 0)),
            pl.BlockSpec((DM, H * D), lambda i: (0, 0)),
            pl.BlockSpec((DM, H * D), lambda i: (0, 0)),
            pl.BlockSpec((DM, D), lambda i: (0, 0)),
            pl.BlockSpec((DM, D), lambda i: (0, 0)),
            pl.BlockSpec((DM, H), lambda i: (0, 0)),
            pl.BlockSpec((QBLK, H * D), lambda i: (i, 0)),
            pl.BlockSpec((QBLK, H * D), lambda i: (i, 0)),
            pl.BlockSpec((QBLK, D), lambda i: (i, 0)),
            pl.BlockSpec((QBLK, D), lambda i: (i, 0)),
        ],
        out_specs=[
            pl.BlockSpec((QBLK, H * D), lambda i: (i, 0)),
            pl.BlockSpec((QBLK, D), lambda i: (i, 0)),
            pl.BlockSpec((QBLK, H), lambda i: (i, 0)),
        ],
        out_shape=[
            jax.ShapeDtypeStruct((S, H * D), jnp.float32),
            jax.ShapeDtypeStruct((S, D), jnp.float32),
            jax.ShapeDtypeStruct((S, H), jnp.float32),
        ],
    )(hs, wq, wq_sw, wk, wk_sw, ww, c2h, s2h, c2, s2)

    scores = pl.pallas_call(
        _scores_body,
        grid=(grid,),
        in_specs=[
            pl.BlockSpec((QBLK, H * D), lambda i: (i, 0)),
            pl.BlockSpec((QBLK, H), lambda i: (i, 0)),
            pl.BlockSpec((S, D), lambda i: (0, 0)),
        ],
        out_specs=pl.BlockSpec((QBLK, S), lambda i: (i, 0)),
        out_shape=jax.ShapeDtypeStruct((S, S), jnp.float32),
    )(q, w, k)
    return scores


def _rev(x):
    return lax.rev(x, (0,))


def _ce_reg(ka, ia, kb, ib):
    m = ka >= kb
    return (jnp.where(m, ka, kb), jnp.where(m, ia, ib),
            jnp.where(m, kb, ka), jnp.where(m, ib, ia))


def _vsort_desc(k, i):
    return plsc.sort_key_val(k, i, descending=True)


def _bitonic_desc_reg(K, I, base, nv):
    dv = nv // 2
    while dv >= 1:
        for g in range(0, nv, 2 * dv):
            for t in range(dv):
                p, q = base + g + t, base + g + t + dv
                K[p], I[p], K[q], I[q] = _ce_reg(K[p], I[p], K[q], I[q])
        dv //= 2
    for p in range(base, base + nv):
        K[p], I[p] = _vsort_desc(K[p], I[p])


def _merge_reg(K, I, a, nv):
    for j in range(nv):
        p, q = a + j, a + 2 * nv - 1 - j
        hi_k, hi_i, lo_k, lo_i = _ce_reg(K[p], I[p], _rev(K[q]), _rev(I[q]))
        K[p], I[p] = hi_k, hi_i
        K[q], I[q] = _rev(lo_k), _rev(lo_i)
    _bitonic_desc_reg(K, I, a, nv)
    _bitonic_desc_reg(K, I, a + nv, nv)


def _sort_chunk_reg(K, I):
    for p in range(NVC):
        K[p], I[p] = _vsort_desc(K[p], I[p])
    nv = 1
    while nv < NVC:
        for a in range(0, NVC, 2 * nv):
            _merge_reg(K, I, a, nv)
        nv *= 2


def _ce_mem(keys, idxs, pa, pb, rev_b=False):
    ka = keys[pl.ds(pa, L)]
    ia = idxs[pl.ds(pa, L)]
    kb = keys[pl.ds(pb, L)]
    ib = idxs[pl.ds(pb, L)]
    if rev_b:
        kb, ib = _rev(kb), _rev(ib)
    hk, hi, lk, li = _ce_reg(ka, ia, kb, ib)
    if rev_b:
        lk, li = _rev(lk), _rev(li)
    keys[pl.ds(pa, L)] = hk
    idxs[pl.ds(pa, L)] = hi
    keys[pl.ds(pb, L)] = lk
    idxs[pl.ds(pb, L)] = li


def _vsort_mem(keys, idxs, p):
    k = keys[pl.ds(p, L)]
    i = idxs[pl.ds(p, L)]
    k, i = _vsort_desc(k, i)
    keys[pl.ds(p, L)] = k
    idxs[pl.ds(p, L)] = i


def _bitonic_desc_mem(keys, idxs, base, n):
    d = n // 2
    while d >= L:
        for g in range(0, n, 2 * d):
            for t in range(0, d, L):
                _ce_mem(keys, idxs, base + g + t, base + g + t + d)
        d //= 2
    for p in range(0, n, L):
        _vsort_mem(keys, idxs, base + p)


def _sc_topk(scores):
    mesh = plsc.VectorSubcoreMesh(core_axis_name="c", subcore_axis_name="s",
                                  num_cores=NC, num_subcores=NS)

    @functools.partial(
        pl.kernel,
        out_type=[jax.ShapeDtypeStruct((S, TOPK), jnp.float32),
                  jax.ShapeDtypeStruct((S, TOPK), jnp.int32)],
        mesh=mesh,
        scratch_types=[pltpu.VMEM((S,), jnp.float32),
                       pltpu.VMEM((S,), jnp.float32),
                       pltpu.VMEM((S,), jnp.int32),
                       pltpu.VMEM((TOPK,), jnp.float32)],
        compiler_params=pltpu.CompilerParams(needs_layout_passes=False),
    )
    def topk_kernel(scores_hbm, outv_hbm, outi_hbm, rowbuf, keys, idxs, valbuf):
        wid = lax.axis_index("s") * NC + lax.axis_index("c")
        lanes = lax.iota(jnp.int32, L)

        def row_body(j, carry):
            row = wid + NW * j
            pltpu.sync_copy(scores_hbm.at[row], rowbuf)

            def chunk_body(c, carry2):
                base = c * CHUNK
                K, I = [], []
                for t in range(NVC):
                    off = base + t * L
                    col = lanes + off
                    colf = col.astype(jnp.float32)
                    x = rowbuf[pl.ds(off, L)]
                    key = jnp.where(
                        x > 0.0, x,
                        jnp.where(col <= row, -colf,
                                  -float(MASK_KEY_BIAS) - colf))
                    K.append(key)
                    I.append(col)
                _sort_chunk_reg(K, I)
                for t in range(NVC):
                    keys[pl.ds(base + t * L, L)] = K[t]
                    idxs[pl.ds(base + t * L, L)] = I[t]
                return carry2

            lax.fori_loop(0, S // CHUNK, chunk_body, 0, unroll=False)

            def merge512_body(m, carry2):
                a = m * 2 * CHUNK
                for j in range(0, CHUNK, L):
                    _ce_mem(keys, idxs, a + j, a + 2 * CHUNK - L - j,
                            rev_b=True)
                _bitonic_desc_mem(keys, idxs, a, CHUNK)
                _bitonic_desc_mem(keys, idxs, a + CHUNK, CHUNK)
                return carry2

            lax.fori_loop(0, S // (2 * CHUNK), merge512_body, 0, unroll=False)

            def prune_body(g, carry2):
                b = g * TOPK
                for j in range(0, TOPK, L):
                    pa = j
                    pb = b + TOPK - L - j
                    ka = keys[pl.ds(pa, L)]
                    ia = idxs[pl.ds(pa, L)]
                    kb = _rev(keys[pl.ds(pb, L)])
                    ib = _rev(idxs[pl.ds(pb, L)])
                    m = ka >= kb
                    keys[pl.ds(pa, L)] = jnp.where(m, ka, kb)
                    idxs[pl.ds(pa, L)] = jnp.where(m, ia, ib)
                _bitonic_desc_mem(keys, idxs, 0, TOPK)
                return carry2

            lax.fori_loop(1, S // TOPK, prune_body, 0, unroll=False)

            for t in range(TOPK // L):
                kk = keys[pl.ds(t * L, L)]
                val = jnp.where(
                    kk > 0.0, kk,
                    jnp.where(kk > -float(MASK_KEY_BIAS), 0.0, MASK_VAL))
                valbuf[pl.ds(t * L, L)] = val
            pltpu.sync_copy(valbuf, outv_hbm.at[row])
            pltpu.sync_copy(idxs.at[pl.ds(0, TOPK)], outi_hbm.at[row])
            return carry

        lax.fori_loop(0, ROWS_PER_W, row_body, 0, unroll=False)

    return topk_kernel(scores)


@jax.jit
def kernel(hidden_states, cos, sin, wq, wk, ww):
    hs = hidden_states[0]
    scores = _compute_scores(hs, cos[0], sin[0], wq, wk, ww)
    tv, ti = _sc_topk(scores)
    return tv[None], ti[None]


# speedup vs baseline: 2.4434x; 1.2186x over previous
import functools

import jax
import jax.numpy as jnp
import numpy as np
from jax import lax
from jax.experimental import pallas as pl
from jax.experimental.pallas import tpu as pltpu
from jax.experimental.pallas import tpu_sc as plsc

S = 2048
DM = 2048
H = 12
D = 64
RD = 32
TOPK = 512
QBLK = 256

NC = 2
NS = 16
NW = NC * NS
ROWS_PER_W = S // NW
L = 16
CHUNK = 256
NVC = CHUNK // L

MASK_VAL = -1e9
ZERO_KEY_BIAS = 0
MASK_KEY_BIAS = 4096


def _proj_body(hs_ref, wq_ref, wqs_ref, wk_ref, wks_ref, ww_ref,
               c2h_ref, s2h_ref, c2_ref, s2_ref, q_ref, k_ref, w_ref):
    hsb = hs_ref[...]
    q1 = jnp.dot(hsb, wq_ref[...], preferred_element_type=jnp.float32)
    q2 = jnp.dot(hsb, wqs_ref[...], preferred_element_type=jnp.float32)
    q_ref[...] = q1 * c2h_ref[...] + q2 * s2h_ref[...]
    k1 = jnp.dot(hsb, wk_ref[...], preferred_element_type=jnp.float32)
    k2 = jnp.dot(hsb, wks_ref[...], preferred_element_type=jnp.float32)
    k_ref[...] = k1 * c2_ref[...] + k2 * s2_ref[...]
    g = jnp.dot(hsb, ww_ref[...], preferred_element_type=jnp.float32)
    g = g - jnp.max(g, axis=-1, keepdims=True)
    e = jnp.exp(g)
    w_ref[...] = e / jnp.sum(e, axis=-1, keepdims=True)


def _scores_body(q_ref, w_ref, k_ref, out_ref):
    i = pl.program_id(0)
    q = q_ref[...]
    k = k_ref[...]
    w = w_ref[...]
    acc = jnp.zeros((QBLK, S), dtype=jnp.float32)
    for h in range(H):
        qh = q[:, h * D:(h + 1) * D]
        lh = lax.dot_general(qh, k, (((1,), (1,)), ((), ())),
                             preferred_element_type=jnp.float32)
        acc = acc + jnp.maximum(lh * (D ** -0.5), 0.0) * w[:, h:h + 1]
    row = i * QBLK + lax.broadcasted_iota(jnp.int32, (QBLK, S), 0)
    col = lax.broadcasted_iota(jnp.int32, (QBLK, S), 1)
    out_ref[...] = jnp.where(col <= row, acc, MASK_VAL)


def _compute_scores(hs, cos, sin, wq, wk, ww):
    c = cos[:, :RD // 2]
    sn = sin[:, :RD // 2]
    c2r = jnp.repeat(c, 2, axis=-1)
    s2r = jnp.repeat(sn, 2, axis=-1)
    sgn = jnp.tile(jnp.array([-1.0, 1.0], dtype=jnp.float32), RD // 2)
    ones = jnp.ones((S, D - RD), dtype=jnp.float32)
    zeros = jnp.zeros((S, D - RD), dtype=jnp.float32)
    c2 = jnp.concatenate([ones, c2r], axis=-1)
    s2 = jnp.concatenate([zeros, s2r * sgn], axis=-1)
    c2h = jnp.tile(c2, (1, H))
    s2h = jnp.tile(s2, (1, H))
    perm = np.arange(D)
    perm[D - RD:] = perm[D - RD:] ^ 1
    wq_sw = wq.reshape(DM, H, D)[:, :, perm].reshape(DM, H * D)
    wk_sw = wk[:, perm]

    grid = S // QBLK
    q, k, w = pl.pallas_call(
        _proj_body,
        grid=(grid,),
        in_specs=[
            pl.BlockSpec((QBLK, DM), lambda i: (i, 0)),
            pl.BlockSpec((DM, H * D), lambda i: (0, 0)),
            pl.BlockSpec((DM, H * D), lambda i: (0, 0)),
            pl.BlockSpec((DM, D), lambda i: (0, 0)),
            pl.BlockSpec((DM, D), lambda i: (0, 0)),
            pl.BlockSpec((DM, H), lambda i: (0, 0)),
            pl.BlockSpec((QBLK, H * D), lambda i: (i, 0)),
            pl.BlockSpec((QBLK, H * D), lambda i: (i, 0)),
            pl.BlockSpec((QBLK, D), lambda i: (i, 0)),
            pl.BlockSpec((QBLK, D), lambda i: (i, 0)),
        ],
        out_specs=[
            pl.BlockSpec((QBLK, H * D), lambda i: (i, 0)),
            pl.BlockSpec((QBLK, D), lambda i: (i, 0)),
            pl.BlockSpec((QBLK, H), lambda i: (i, 0)),
        ],
        out_shape=[
            jax.ShapeDtypeStruct((S, H * D), jnp.float32),
            jax.ShapeDtypeStruct((S, D), jnp.float32),
            jax.ShapeDtypeStruct((S, H), jnp.float32),
        ],
    )(hs, wq, wq_sw, wk, wk_sw, ww, c2h, s2h, c2, s2)

    scores = pl.pallas_call(
        _scores_body,
        grid=(grid,),
        in_specs=[
            pl.BlockSpec((QBLK, H * D), lambda i: (i, 0)),
            pl.BlockSpec((QBLK, H), lambda i: (i, 0)),
            pl.BlockSpec((S, D), lambda i: (0, 0)),
        ],
        out_specs=pl.BlockSpec((QBLK, S), lambda i: (i, 0)),
        out_shape=jax.ShapeDtypeStruct((S, S), jnp.float32),
    )(q, w, k)
    return scores


def _rev(x):
    return lax.rev(x, (0,))


def _ce_reg(ka, ia, kb, ib):
    m = ka >= kb
    return (jnp.where(m, ka, kb), jnp.where(m, ia, ib),
            jnp.where(m, kb, ka), jnp.where(m, ib, ia))


def _vsort_desc(k, i):
    return plsc.sort_key_val(k, i, descending=True)


def _bitonic_desc_reg(K, I, base, nv):
    dv = nv // 2
    while dv >= 1:
        for g in range(0, nv, 2 * dv):
            for t in range(dv):
                p, q = base + g + t, base + g + t + dv
                K[p], I[p], K[q], I[q] = _ce_reg(K[p], I[p], K[q], I[q])
        dv //= 2
    for p in range(base, base + nv):
        K[p], I[p] = _vsort_desc(K[p], I[p])


def _merge_reg(K, I, a, nv):
    for j in range(nv):
        p, q = a + j, a + 2 * nv - 1 - j
        hi_k, hi_i, lo_k, lo_i = _ce_reg(K[p], I[p], _rev(K[q]), _rev(I[q]))
        K[p], I[p] = hi_k, hi_i
        K[q], I[q] = _rev(lo_k), _rev(lo_i)
    _bitonic_desc_reg(K, I, a, nv)
    _bitonic_desc_reg(K, I, a + nv, nv)


def _sort_chunk_reg(K, I):
    for p in range(NVC):
        K[p], I[p] = _vsort_desc(K[p], I[p])
    nv = 1
    while nv < NVC:
        for a in range(0, NVC, 2 * nv):
            _merge_reg(K, I, a, nv)
        nv *= 2


def _ce_mem(keys, idxs, pa, pb, rev_b=False):
    ka = keys[pl.ds(pa, L)]
    ia = idxs[pl.ds(pa, L)]
    kb = keys[pl.ds(pb, L)]
    ib = idxs[pl.ds(pb, L)]
    if rev_b:
        kb, ib = _rev(kb), _rev(ib)
    hk, hi, lk, li = _ce_reg(ka, ia, kb, ib)
    if rev_b:
        lk, li = _rev(lk), _rev(li)
    keys[pl.ds(pa, L)] = hk
    idxs[pl.ds(pa, L)] = hi
    keys[pl.ds(pb, L)] = lk
    idxs[pl.ds(pb, L)] = li


def _vsort_mem(keys, idxs, p):
    k = keys[pl.ds(p, L)]
    i = idxs[pl.ds(p, L)]
    k, i = _vsort_desc(k, i)
    keys[pl.ds(p, L)] = k
    idxs[pl.ds(p, L)] = i


def _bitonic_desc_mem(keys, idxs, base, n):
    d = n // 2
    while d >= L:
        for g in range(0, n, 2 * d):
            for t in range(0, d, L):
                _ce_mem(keys, idxs, base + g + t, base + g + t + d)
        d //= 2
    for p in range(0, n, L):
        _vsort_mem(keys, idxs, base + p)


def _sc_topk(scores):
    mesh = plsc.VectorSubcoreMesh(core_axis_name="c", subcore_axis_name="s",
                                  num_cores=NC, num_subcores=NS)

    @functools.partial(
        pl.kernel,
        out_type=[jax.ShapeDtypeStruct((S, TOPK), jnp.float32),
                  jax.ShapeDtypeStruct((S, TOPK), jnp.int32)],
        mesh=mesh,
        scratch_types=[pltpu.VMEM((S,), jnp.float32),
                       pltpu.VMEM((S,), jnp.float32),
                       pltpu.VMEM((S,), jnp.int32),
                       pltpu.VMEM((TOPK,), jnp.float32)],
        compiler_params=pltpu.CompilerParams(needs_layout_passes=False),
    )
    def topk_kernel(scores_hbm, outv_hbm, outi_hbm, rowbuf, keys, idxs, valbuf):
        wid = lax.axis_index("s") * NC + lax.axis_index("c")
        lanes = lax.iota(jnp.int32, L)

        def row_body(j, carry):
            row = wid + NW * j
            ng = row // TOPK + 1
            pltpu.sync_copy(scores_hbm.at[row], rowbuf)

            def chunk_body(c, carry2):
                base = c * CHUNK
                K, I = [], []
                for t in range(NVC):
                    off = base + t * L
                    col = lanes + off
                    colf = col.astype(jnp.float32)
                    x = rowbuf[pl.ds(off, L)]
                    key = jnp.where(
                        x > 0.0, x,
                        jnp.where(col <= row, -colf,
                                  -float(MASK_KEY_BIAS) - colf))
                    K.append(key)
                    I.append(col)
                _sort_chunk_reg(K, I)
                for t in range(NVC):
                    keys[pl.ds(base + t * L, L)] = K[t]
                    idxs[pl.ds(base + t * L, L)] = I[t]
                return carry2

            lax.fori_loop(0, 2 * ng, chunk_body, 0, unroll=False)

            def merge512_body(m, carry2):
                a = m * 2 * CHUNK
                for j in range(0, CHUNK, L):
                    _ce_mem(keys, idxs, a + j, a + 2 * CHUNK - L - j,
                            rev_b=True)
                _bitonic_desc_mem(keys, idxs, a, CHUNK)
                _bitonic_desc_mem(keys, idxs, a + CHUNK, CHUNK)
                return carry2

            lax.fori_loop(0, ng, merge512_body, 0, unroll=False)

            def prune_body(g, carry2):
                b = g * TOPK
                for j in range(0, TOPK, L):
                    pa = j
                    pb = b + TOPK - L - j
                    ka = keys[pl.ds(pa, L)]
                    ia = idxs[pl.ds(pa, L)]
                    kb = _rev(keys[pl.ds(pb, L)])
                    ib = _rev(idxs[pl.ds(pb, L)])
                    m = ka >= kb
                    keys[pl.ds(pa, L)] = jnp.where(m, ka, kb)
                    idxs[pl.ds(pa, L)] = jnp.where(m, ia, ib)
                _bitonic_desc_mem(keys, idxs, 0, TOPK)
                return carry2

            lax.fori_loop(1, ng, prune_body, 0, unroll=False)

            for t in range(TOPK // L):
                kk = keys[pl.ds(t * L, L)]
                val = jnp.where(
                    kk > 0.0, kk,
                    jnp.where(kk > -float(MASK_KEY_BIAS), 0.0, MASK_VAL))
                valbuf[pl.ds(t * L, L)] = val
            pltpu.sync_copy(valbuf, outv_hbm.at[row])
            pltpu.sync_copy(idxs.at[pl.ds(0, TOPK)], outi_hbm.at[row])
            return carry

        lax.fori_loop(0, ROWS_PER_W, row_body, 0, unroll=False)

    return topk_kernel(scores)


@jax.jit
def kernel(hidden_states, cos, sin, wq, wk, ww):
    hs = hidden_states[0]
    scores = _compute_scores(hs, cos[0], sin[0], wq, wk, ww)
    tv, ti = _sc_topk(scores)
    return tv[None], ti[None]
